# baseline (speedup 1.0000x reference)
import functools

import jax
import jax.numpy as jnp
from jax import lax
from jax.experimental import pallas as pl
from jax.experimental.pallas import tpu as pltpu
from jax.experimental.pallas import tpu_sc as plsc

N_BBLK = 8
N_LGRP = 4
LANE = 128
VEC = 16


def kernel(idx, targets, token_embedding_table):
    del targets
    B, L = idx.shape
    V, D = token_embedding_table.shape
    n_strip = (D + LANE - 1) // LANE
    n_full = D // LANE
    rem = D - n_full * LANE

    tabs = (
        jnp.pad(token_embedding_table, ((0, 0), (0, n_strip * LANE - D)))
        .reshape(V, n_strip, LANE)
        .transpose(1, 0, 2)
    )
    idx_t = idx.astype(jnp.int32).T.reshape(L, N_BBLK, LANE)

    mesh = plsc.VectorSubcoreMesh(core_axis_name="c", subcore_axis_name="s")

    @functools.partial(
        pl.kernel,
        out_type=jax.ShapeDtypeStruct((L, D, B), jnp.float32),
        mesh=mesh,
        compiler_params=pltpu.CompilerParams(needs_layout_passes=False),
        scratch_types=[
            pltpu.VMEM((LANE,), jnp.int32),
            pltpu.VMEM((LANE, LANE), jnp.float32),
            pltpu.VMEM((LANE, LANE), jnp.float32),
            pltpu.SemaphoreType.DMA,
            pltpu.SemaphoreType.DMA,
        ],
    )
    def gather_kernel(tabs_hbm, idx_hbm, out_hbm, idx_v, buf_in, buf_out, gs, ws):
        wid = lax.axis_index("s") * 2 + lax.axis_index("c")
        bblk = wid % N_BBLK
        lg = wid // N_BBLK

        rows = [lax.iota(jnp.int32, VEC) + k * VEC for k in range(LANE // VEC)]

        def transpose_tile():
            @pl.loop(0, LANE)
            def _per_d(d):
                col = jnp.full((VEC,), d, jnp.int32)
                for k in range(LANE // VEC):
                    buf_out[d, pl.ds(k * VEC, VEC)] = plsc.load_gather(
                        buf_in, [rows[k], col]
                    )

        for c in range(n_strip):
            d_lo = c * LANE
            d_sz = LANE if c < n_full else rem

            @pl.loop(lg, L, step=N_LGRP)
            def _per_l(l):
                pltpu.sync_copy(idx_hbm.at[l, bblk], idx_v)
                pltpu.async_copy(tabs_hbm.at[c].at[idx_v], buf_in, gs).wait()
                transpose_tile()
                pltpu.async_copy(
                    buf_out.at[pl.ds(0, d_sz)],
                    out_hbm.at[l, pl.ds(d_lo, d_sz), pl.ds(bblk * LANE, LANE)],
                    ws,
                ).wait()

    out = gather_kernel(tabs, idx_t)
    return out.transpose(2, 0, 1)
